# baseline (speedup 1.0000x reference)
import functools

import jax
import jax.numpy as jnp
from jax import lax
from jax.experimental import pallas as pl
from jax.experimental.pallas import tpu as pltpu
from jax.experimental.pallas import tpu_sc as plsc

_H, _W = 60, 80
_HW = _H * _W
_QB = 480
_K = 3
_D = 128
_NC, _NS = 2, 16


def _knn_body(gxy_ref, uv_ref, idx_ref):
    N = uv_ref.shape[2]
    b = pl.program_id(0)
    qx = gxy_ref[:, 0:1]
    qy = gxy_ref[:, 1:2]
    px = uv_ref[0, 0:1, :]
    py = uv_ref[0, 1:2, :]
    q2 = qx * qx + qy * qy
    p2 = px * px + py * py
    qp = jnp.dot(gxy_ref[:, :], uv_ref[0],
                 preferred_element_type=jnp.float32)
    d = q2 - 2.0 * qp + p2

    lif = jax.lax.broadcasted_iota(jnp.int32, (_QB, N), 1).astype(jnp.float32)
    cols = []
    for k in range(_K):
        m = jnp.min(d, axis=1, keepdims=True)
        idxf = jnp.min(jnp.where(d == m, lif, jnp.float32(N)), axis=1,
                       keepdims=True)
        cols.append(idxf.astype(jnp.int32))
        if k < _K - 1:
            d = jnp.where(lif == idxf, jnp.inf, d)
    idx_ref[0] = jnp.concatenate(cols, axis=1) + b * N


def _interp_body(gxyt_ref, g0_ref, g1_ref, g2_ref, w1_ref, b1_ref, w2_ref,
                 b2_ref, wo_ref, bo_ref, out_ref):
    C = w2_ref.shape[0]
    qx = gxyt_ref[0:1, :]
    qy = gxyt_ref[1:2, :]
    acc = None
    for g_ref in (g0_ref, g1_ref, g2_ref):
        gt = g_ref[:, :].T
        offx = gt[0:1, :] - qx
        offy = gt[1:2, :] - qy
        nrm = jnp.sqrt(offx * offx + offy * offy)
        logits = b2_ref[:, 0:1]
        for j in range(3):
            h = (offx * w1_ref[j:j + 1, 0:1] + offy * w1_ref[j:j + 1, 1:2]
                 + nrm * w1_ref[j:j + 1, 2:3] + b1_ref[0:1, j:j + 1])
            h = jnp.where(h >= 0, h, 0.1 * h)
            logits = logits + w2_ref[:, j:j + 1] * h
        score = jax.nn.sigmoid(logits)
        term = score * gt[2:2 + C, :]
        acc = term if acc is None else acc + term
    o = jnp.dot(wo_ref[:, :], acc, preferred_element_type=jnp.float32,
                precision=jax.lax.Precision.HIGHEST) + bo_ref[:, 0:1]
    out_ref[0] = jnp.where(o >= 0, o, 0.1 * o)


def _sc_gather(table, idx_flat, b_per_w):
    nw = _NC * _NS
    mesh = plsc.VectorSubcoreMesh(core_axis_name="c", subcore_axis_name="s")

    h = b_per_w // 2

    @functools.partial(
        pl.kernel, mesh=mesh,
        out_type=jax.ShapeDtypeStruct((b_per_w * nw, _D), jnp.float32),
        scratch_types=[
            pltpu.VMEM((h,), jnp.int32),
            pltpu.VMEM((h,), jnp.int32),
            pltpu.VMEM((h, _D), jnp.float32),
            pltpu.VMEM((h, _D), jnp.float32),
            pltpu.SemaphoreType.DMA,
            pltpu.SemaphoreType.DMA,
            pltpu.SemaphoreType.DMA,
            pltpu.SemaphoreType.DMA,
        ],
    )
    def k(idx_hbm, table_hbm, out_hbm, idx0_v, idx1_v, rows0_v, rows1_v,
          g0s, g1s, w0s, w1s):
        wid = lax.axis_index("s") * _NC + lax.axis_index("c")
        base = wid * b_per_w
        pltpu.sync_copy(idx_hbm.at[pl.ds(base, h)], idx0_v)
        pltpu.sync_copy(idx_hbm.at[pl.ds(base + h, h)], idx1_v)
        g0 = pltpu.async_copy(table_hbm.at[idx0_v], rows0_v, g0s)
        g1 = pltpu.async_copy(table_hbm.at[idx1_v], rows1_v, g1s)
        g0.wait()
        w0 = pltpu.async_copy(rows0_v, out_hbm.at[pl.ds(base, h)], w0s)
        g1.wait()
        w1 = pltpu.async_copy(rows1_v, out_hbm.at[pl.ds(base + h, h)], w1s)
        w0.wait()
        w1.wait()

    return k(idx_flat, table)


def kernel(uv, feat_3d, w1, b1, w2, b2, w_out, b_out, image_h, image_w):
    bs, _, N = uv.shape
    C = feat_3d.shape[1]
    T = _HW // _QB

    r = ((jnp.asarray(image_h, jnp.float32) - _H)
         + (jnp.asarray(image_w, jnp.float32) - _W))
    idx = jnp.arange(_HW, dtype=jnp.int32)
    xs = (idx % _W).astype(jnp.float32)
    ys = (idx // _W).astype(jnp.float32)
    gxy = jnp.stack([xs, ys], axis=1) + r

    knn_idx = pl.pallas_call(
        _knn_body,
        grid=(bs, T),
        in_specs=[
            pl.BlockSpec((_QB, 2), lambda b, t: (t, 0)),
            pl.BlockSpec((1, 2, N), lambda b, t: (b, 0, 0)),
        ],
        out_specs=pl.BlockSpec((1, _QB, _K), lambda b, t: (b, t, 0)),
        out_shape=jax.ShapeDtypeStruct((bs, _HW, _K), jnp.int32),
    )(gxy, uv)

    table = jnp.concatenate(
        [jnp.swapaxes(uv, 1, 2), jnp.swapaxes(feat_3d, 1, 2),
         jnp.zeros((bs, N, _D - 2 - C), jnp.float32)], axis=2,
    ).reshape(bs * N, _D)
    B = bs * _HW * _K
    nw = _NC * _NS
    b_pad = (B + 16 * nw - 1) // (16 * nw) * (16 * nw)
    idx_flat = jnp.concatenate(
        [jnp.transpose(knn_idx, (2, 0, 1)).reshape(B),
         jnp.zeros((b_pad - B,), jnp.int32)])
    rows = _sc_gather(table, idx_flat, b_pad // nw)

    out_qm = pl.pallas_call(
        _interp_body,
        grid=(bs,),
        in_specs=[
            pl.BlockSpec((2, _HW), lambda b: (0, 0)),
            pl.BlockSpec((_HW, _D), lambda b: (0 * bs + b, 0)),
            pl.BlockSpec((_HW, _D), lambda b: (1 * bs + b, 0)),
            pl.BlockSpec((_HW, _D), lambda b: (2 * bs + b, 0)),
            pl.BlockSpec((3, 3), lambda b: (0, 0)),
            pl.BlockSpec((1, 3), lambda b: (0, 0)),
            pl.BlockSpec((C, 3), lambda b: (0, 0)),
            pl.BlockSpec((C, 1), lambda b: (0, 0)),
            pl.BlockSpec((C, C), lambda b: (0, 0)),
            pl.BlockSpec((C, 1), lambda b: (0, 0)),
        ],
        out_specs=pl.BlockSpec((1, C, _HW), lambda b: (b, 0, 0)),
        out_shape=jax.ShapeDtypeStruct((bs, C, _HW), jnp.float32),
    )(gxy.T, rows, rows, rows, w1, b1[None, :], w2, b2[:, None], w_out,
      b_out[:, None])

    return out_qm.reshape(bs, C, _H, _W)


# speedup vs baseline: 1.0762x; 1.0762x over previous
import functools

import jax
import jax.numpy as jnp
from jax import lax
from jax.experimental import pallas as pl
from jax.experimental.pallas import tpu as pltpu
from jax.experimental.pallas import tpu_sc as plsc

_H, _W = 60, 80
_HW = _H * _W
_QB = 600
_K = 3
_D = 128
_NC, _NS = 2, 16


def _knn_body(gxy_ref, uv_ref, idx_ref):
    N = uv_ref.shape[2]
    b = pl.program_id(0)
    qx = gxy_ref[:, 0:1]
    qy = gxy_ref[:, 1:2]
    px = uv_ref[0, 0:1, :]
    py = uv_ref[0, 1:2, :]
    q2 = qx * qx + qy * qy
    p2 = px * px + py * py
    qp = jnp.dot(gxy_ref[:, :], uv_ref[0],
                 preferred_element_type=jnp.float32)
    d = q2 - 2.0 * qp + p2

    lif = jax.lax.broadcasted_iota(jnp.int32, (_QB, N), 1).astype(jnp.float32)
    cols = []
    for k in range(_K):
        m = jnp.min(d, axis=1, keepdims=True)
        idxf = jnp.min(jnp.where(d == m, lif, jnp.float32(N)), axis=1,
                       keepdims=True)
        cols.append(idxf.astype(jnp.int32))
        if k < _K - 1:
            d = jnp.where(lif == idxf, jnp.inf, d)
    idx_ref[0] = jnp.concatenate(cols, axis=1) + b * N


def _interp_body(gxyt_ref, g0_ref, g1_ref, g2_ref, w1_ref, b1_ref, w2_ref,
                 b2_ref, wo_ref, bo_ref, out_ref):
    C = w2_ref.shape[0]
    qx = gxyt_ref[0:1, :]
    qy = gxyt_ref[1:2, :]
    acc = None
    for g_ref in (g0_ref, g1_ref, g2_ref):
        gt = g_ref[:, :].T
        offx = gt[0:1, :] - qx
        offy = gt[1:2, :] - qy
        nrm = jnp.sqrt(offx * offx + offy * offy)
        logits = b2_ref[:, 0:1]
        for j in range(3):
            h = (offx * w1_ref[j:j + 1, 0:1] + offy * w1_ref[j:j + 1, 1:2]
                 + nrm * w1_ref[j:j + 1, 2:3] + b1_ref[0:1, j:j + 1])
            h = jnp.where(h >= 0, h, 0.1 * h)
            logits = logits + w2_ref[:, j:j + 1] * h
        score = jax.nn.sigmoid(logits)
        term = score * gt[2:2 + C, :]
        acc = term if acc is None else acc + term
    o = jnp.dot(wo_ref[:, :], acc, preferred_element_type=jnp.float32,
                precision=jax.lax.Precision.HIGHEST) + bo_ref[:, 0:1]
    out_ref[0] = jnp.where(o >= 0, o, 0.1 * o)


def _sc_gather(table, idx_flat, b_per_w):
    nw = _NC * _NS
    mesh = plsc.VectorSubcoreMesh(core_axis_name="c", subcore_axis_name="s")

    @functools.partial(
        pl.kernel, mesh=mesh,
        out_type=jax.ShapeDtypeStruct((b_per_w * nw, _D), jnp.float32),
        scratch_types=[
            pltpu.VMEM((b_per_w,), jnp.int32),
            pltpu.VMEM((b_per_w, _D), jnp.float32),
            pltpu.SemaphoreType.DMA,
        ],
    )
    def k(idx_hbm, table_hbm, out_hbm, idx_v, rows_v, sem):
        wid = lax.axis_index("s") * _NC + lax.axis_index("c")
        base = wid * b_per_w
        pltpu.sync_copy(idx_hbm.at[pl.ds(base, b_per_w)], idx_v)
        pltpu.async_copy(table_hbm.at[idx_v], rows_v, sem).wait()
        pltpu.sync_copy(rows_v, out_hbm.at[pl.ds(base, b_per_w)])

    return k(idx_flat, table)


def kernel(uv, feat_3d, w1, b1, w2, b2, w_out, b_out, image_h, image_w):
    bs, _, N = uv.shape
    C = feat_3d.shape[1]
    T = _HW // _QB

    r = ((jnp.asarray(image_h, jnp.float32) - _H)
         + (jnp.asarray(image_w, jnp.float32) - _W))
    idx = jnp.arange(_HW, dtype=jnp.int32)
    xs = (idx % _W).astype(jnp.float32)
    ys = (idx // _W).astype(jnp.float32)
    gxy = jnp.stack([xs, ys], axis=1) + r

    knn_idx = pl.pallas_call(
        _knn_body,
        grid=(bs, T),
        in_specs=[
            pl.BlockSpec((_QB, 2), lambda b, t: (t, 0)),
            pl.BlockSpec((1, 2, N), lambda b, t: (b, 0, 0)),
        ],
        out_specs=pl.BlockSpec((1, _QB, _K), lambda b, t: (b, t, 0)),
        out_shape=jax.ShapeDtypeStruct((bs, _HW, _K), jnp.int32),
    )(gxy, uv)

    table = jnp.concatenate(
        [jnp.swapaxes(uv, 1, 2), jnp.swapaxes(feat_3d, 1, 2),
         jnp.zeros((bs, N, _D - 2 - C), jnp.float32)], axis=2,
    ).reshape(bs * N, _D)
    B = bs * _HW * _K
    nw = _NC * _NS
    b_pad = (B + 8 * nw - 1) // (8 * nw) * (8 * nw)
    idx_flat = jnp.concatenate(
        [jnp.transpose(knn_idx, (2, 0, 1)).reshape(B),
         jnp.zeros((b_pad - B,), jnp.int32)])
    rows = _sc_gather(table, idx_flat, b_pad // nw)

    out_qm = pl.pallas_call(
        _interp_body,
        grid=(bs,),
        in_specs=[
            pl.BlockSpec((2, _HW), lambda b: (0, 0)),
            pl.BlockSpec((_HW, _D), lambda b: (0 * bs + b, 0)),
            pl.BlockSpec((_HW, _D), lambda b: (1 * bs + b, 0)),
            pl.BlockSpec((_HW, _D), lambda b: (2 * bs + b, 0)),
            pl.BlockSpec((3, 3), lambda b: (0, 0)),
            pl.BlockSpec((1, 3), lambda b: (0, 0)),
            pl.BlockSpec((C, 3), lambda b: (0, 0)),
            pl.BlockSpec((C, 1), lambda b: (0, 0)),
            pl.BlockSpec((C, C), lambda b: (0, 0)),
            pl.BlockSpec((C, 1), lambda b: (0, 0)),
        ],
        out_specs=pl.BlockSpec((1, C, _HW), lambda b: (b, 0, 0)),
        out_shape=jax.ShapeDtypeStruct((bs, C, _HW), jnp.float32),
    )(gxy.T, rows, rows, rows, w1, b1[None, :], w2, b2[:, None], w_out,
      b_out[:, None])

    return out_qm.reshape(bs, C, _H, _W)
